# baseline (speedup 1.0000x reference)
import functools
import jax
import jax.numpy as jnp
from jax import lax
from jax.experimental import pallas as pl
from jax.experimental.pallas import tpu as pltpu
from jax.experimental.pallas import tpu_sc as plsc

NC, NS, LANES = 2, 16, 16
CHUNK = 128


@functools.lru_cache(maxsize=None)
def _make_segsum(npad, n_chunks, w):
    nw = NC * NS
    cpw = n_chunks // nw
    G = 16
    ngroups = cpw // G
    rows_per_tile = npad // NS
    wb_chunks = rows_per_tile // CHUNK
    mesh = plsc.VectorSubcoreMesh(core_axis_name="c", subcore_axis_name="s")

    def body(y_hbm, ei_hbm, p_out, eig0, eig1, rows0, rows1, acc,
             sem0, sem1, esem0, esem1):
        ci = lax.axis_index("c")
        si = lax.axis_index("s")
        wid = si * NC + ci
        zeros16 = jnp.zeros((LANES,), jnp.float32)
        rows = (rows0, rows1)
        sems = (sem0, sem1)
        eigs = (eig0, eig1)
        esems = (esem0, esem1)

        @pl.loop(0, CHUNK)
        def _zr(r):
            for j in range(w // LANES):
                rows0[r, pl.ds(j * LANES, LANES)] = zeros16

        row0 = si * rows_per_tile

        @pl.loop(0, wb_chunks)
        def _za(k):
            pltpu.sync_copy(rows0, acc.at[pl.ds(row0 + k * CHUNK, CHUNK)])

        def eload(g, b):
            pltpu.async_copy(ei_hbm.at[pl.ds(wid * cpw + g * G, G)],
                             eigs[b], esems[b])

        def ewait(b):
            pltpu.make_async_copy(ei_hbm.at[pl.ds(0, G)], eigs[b],
                                  esems[b]).wait()

        eload(0, 0)
        plsc.subcore_barrier()

        for g in range(ngroups):
            eb = g % 2
            ewait(eb)
            if g + 1 < ngroups:
                eload(g + 1, 1 - eb)
            eig = eigs[eb]

            def start(k, b, eig=eig):
                pltpu.async_copy(y_hbm.at[eig.at[k, 0]], rows[b], sems[b])

            def wait(b, eig=eig):
                pltpu.make_async_copy(y_hbm.at[eig.at[0, 0]], rows[b],
                                      sems[b]).wait()

            def scat(k, b, eig=eig):
                pltpu.sync_copy(rows[b], acc.at[eig.at[k, 1]], add=True)

            start(0, 0)

            @pl.loop(0, G // 2 - 1)
            def _edges(k2):
                for b in (0, 1):
                    k = k2 * 2 + b
                    wait(b)
                    start(k + 1, 1 - b)
                    scat(k, b)

            wait(0)
            start(G - 1, 1)
            scat(G - 2, 0)
            wait(1)
            scat(G - 1, 1)

        plsc.subcore_barrier()

        @pl.loop(0, wb_chunks)
        def _wb(k):
            r0 = row0 + k * CHUNK
            pltpu.sync_copy(acc.at[pl.ds(r0, CHUNK)], rows0)
            pltpu.sync_copy(rows0, p_out.at[ci, pl.ds(r0, CHUNK)])

    return pl.kernel(
        body,
        out_type=jax.ShapeDtypeStruct((NC, npad, w), jnp.float32),
        mesh=mesh,
        scratch_types=(
            pltpu.VMEM((G, 2, CHUNK), jnp.int32),
            pltpu.VMEM((G, 2, CHUNK), jnp.int32),
            pltpu.VMEM((CHUNK, w), jnp.float32),
            pltpu.VMEM((CHUNK, w), jnp.float32),
            pltpu.VMEM_SHARED((npad, w), jnp.float32),
            pltpu.SemaphoreType.DMA,
            pltpu.SemaphoreType.DMA,
            pltpu.SemaphoreType.DMA,
            pltpu.SemaphoreType.DMA,
        ),
    )


@functools.lru_cache(maxsize=None)
def _make_deg(npad, n_chunks):
    w = 128
    nw = NC * NS
    cpw = n_chunks // nw
    rows_per_tile = npad // NS
    wb_chunks = rows_per_tile // CHUNK
    mesh = plsc.VectorSubcoreMesh(core_axis_name="c", subcore_axis_name="s")

    def body(ei_hbm, d_out, eis_v, rows_v, dacc, sem0, sem1):
        ci = lax.axis_index("c")
        si = lax.axis_index("s")
        wid = si * NC + ci
        zeros16 = jnp.zeros((LANES,), jnp.float32)
        sems = (sem0, sem1)

        @pl.loop(0, CHUNK)
        def _zr(r):
            for j in range(w // LANES):
                rows_v[r, pl.ds(j * LANES, LANES)] = zeros16

        row0 = si * rows_per_tile

        @pl.loop(0, wb_chunks)
        def _za(k):
            pltpu.sync_copy(rows_v, dacc.at[pl.ds(row0 + k * CHUNK, CHUNK)])

        ones16 = jnp.full((LANES,), 1.0, jnp.float32)

        @pl.loop(0, CHUNK)
        def _fo(r):
            rows_v[r, pl.ds(0, LANES)] = ones16

        pltpu.sync_copy(ei_hbm.at[pl.ds(wid * cpw, cpw)], eis_v)

        plsc.subcore_barrier()

        def start(k, b):
            pltpu.async_copy(rows_v, dacc.at[eis_v.at[k, 1]], sems[b],
                             add=True)

        def wait(b):
            pltpu.make_async_copy(rows_v, dacc.at[eis_v.at[0, 1]],
                                  sems[b]).wait()

        start(0, 0)

        @pl.loop(0, cpw // 2 - 1)
        def _edges(k2):
            for b in (0, 1):
                k = k2 * 2 + b
                start(k + 1, 1 - b)
                wait(b)

        start(cpw - 1, 1)
        wait(0)
        wait(1)

        plsc.subcore_barrier()

        @pl.loop(0, wb_chunks)
        def _wb(k):
            r0 = row0 + k * CHUNK
            pltpu.sync_copy(dacc.at[pl.ds(r0, CHUNK)], rows_v)
            pltpu.sync_copy(rows_v, d_out.at[ci, pl.ds(r0, CHUNK)])

    return pl.kernel(
        body,
        out_type=jax.ShapeDtypeStruct((NC, npad, w), jnp.float32),
        mesh=mesh,
        scratch_types=(
            pltpu.VMEM((cpw, 2, CHUNK), jnp.int32),
            pltpu.VMEM((CHUNK, w), jnp.float32),
            pltpu.VMEM_SHARED((npad, w), jnp.float32),
            pltpu.SemaphoreType.DMA,
            pltpu.SemaphoreType.DMA,
        ),
    )


def _mm_first(npad, d, h, r):
    def body(x_ref, wl_ref, wr_ref, y_ref, z_ref):
        xb = x_ref[...]
        y_ref[...] = jnp.dot(xb, wl_ref[...], preferred_element_type=jnp.float32)
        z_ref[...] = jnp.dot(xb, wr_ref[...], preferred_element_type=jnp.float32)

    return pl.pallas_call(
        body,
        grid=(npad // r,),
        in_specs=[
            pl.BlockSpec((r, d), lambda i: (i, 0)),
            pl.BlockSpec((d, h), lambda i: (0, 0)),
            pl.BlockSpec((d, h), lambda i: (0, 0)),
        ],
        out_specs=[
            pl.BlockSpec((r, h), lambda i: (i, 0)),
            pl.BlockSpec((r, h), lambda i: (i, 0)),
        ],
        out_shape=[
            jax.ShapeDtypeStruct((npad, h), jnp.float32),
            jax.ShapeDtypeStruct((npad, h), jnp.float32),
        ],
    )


def _mm_combine(npad, w_in, w_out, r):
    def body(p_ref, d_ref, z_ref, b_ref, wl_ref, wr_ref, y_ref, z2_ref):
        s = p_ref[0] + p_ref[1]
        deg = d_ref[0, :, 0] + d_ref[1, :, 0]
        dc = jnp.maximum(deg, 1.0)
        hh = s / dc[:, None] + b_ref[...] + z_ref[...]
        hh = jnp.maximum(hh, 0.0)
        y_ref[...] = jnp.dot(hh, wl_ref[...], preferred_element_type=jnp.float32)
        z2_ref[...] = jnp.dot(hh, wr_ref[...], preferred_element_type=jnp.float32)

    return pl.pallas_call(
        body,
        grid=(npad // r,),
        in_specs=[
            pl.BlockSpec((NC, r, w_in), lambda i: (0, i, 0)),
            pl.BlockSpec((NC, r, 128), lambda i: (0, i, 0)),
            pl.BlockSpec((r, w_in), lambda i: (i, 0)),
            pl.BlockSpec((1, w_in), lambda i: (0, 0)),
            pl.BlockSpec((w_in, w_out), lambda i: (0, 0)),
            pl.BlockSpec((w_in, w_out), lambda i: (0, 0)),
        ],
        out_specs=[
            pl.BlockSpec((r, w_out), lambda i: (i, 0)),
            pl.BlockSpec((r, w_out), lambda i: (i, 0)),
        ],
        out_shape=[
            jax.ShapeDtypeStruct((npad, w_out), jnp.float32),
            jax.ShapeDtypeStruct((npad, w_out), jnp.float32),
        ],
    )


def _mm_final(npad, w, r):
    def body(p_ref, d_ref, z_ref, b_ref, o_ref):
        s = p_ref[0] + p_ref[1]
        deg = d_ref[0, :, 0] + d_ref[1, :, 0]
        dc = jnp.maximum(deg, 1.0)
        o_ref[...] = s / dc[:, None] + b_ref[...] + z_ref[...]

    return pl.pallas_call(
        body,
        grid=(npad // r,),
        in_specs=[
            pl.BlockSpec((NC, r, w), lambda i: (0, i, 0)),
            pl.BlockSpec((NC, r, 128), lambda i: (0, i, 0)),
            pl.BlockSpec((r, w), lambda i: (i, 0)),
            pl.BlockSpec((1, w), lambda i: (0, 0)),
        ],
        out_specs=pl.BlockSpec((r, w), lambda i: (i, 0)),
        out_shape=jax.ShapeDtypeStruct((npad, w), jnp.float32),
    )


def kernel(x, edge_index, relations, Wl1, bl1, Wr1, Wl2, bl2, Wr2, Wl3, bl3, Wr3):
    n, d = x.shape
    e = edge_index.shape[1]
    h = Wl1.shape[1]
    c = Wl3.shape[1]
    cpad = 128
    tile = NS * CHUNK
    npad = ((n + tile - 1) // tile) * tile
    if npad == n:
        npad += tile
    r = 1024

    x_pad = jnp.pad(x, ((0, npad - n), (0, 0)))
    nchunks = e // CHUNK
    nw = NC * NS
    ncp = -(-nchunks // (16 * nw)) * (16 * nw)
    ei = edge_index.astype(jnp.int32).reshape(2, nchunks, CHUNK)
    ei = jnp.transpose(ei, (1, 0, 2))
    if ncp > nchunks:
        npc = ncp - nchunks
        spread = n + (jnp.arange(npc * CHUNK, dtype=jnp.int32) % (npad - n))
        pad = jnp.stack(
            [jnp.zeros((npc, CHUNK), jnp.int32),
             spread.reshape(npc, CHUNK)], axis=1)
        ei = jnp.concatenate([ei, pad], axis=0)
    wl3p = jnp.pad(Wl3, ((0, 0), (0, cpad - c)))
    wr3p = jnp.pad(Wr3, ((0, 0), (0, cpad - c)))
    bl3p = jnp.pad(bl3, (0, cpad - c)).reshape(1, cpad)

    dpart = _make_deg(npad, ncp)(ei)
    y1, z1 = _mm_first(npad, d, h, r)(x_pad, Wl1, Wr1)
    p1 = _make_segsum(npad, ncp, h)(y1, ei)
    y2, z2 = _mm_combine(npad, h, h, r)(p1, dpart, z1, bl1.reshape(1, h), Wl2, Wr2)
    p2 = _make_segsum(npad, ncp, h)(y2, ei)
    y3, z3 = _mm_combine(npad, h, cpad, r)(p2, dpart, z2, bl2.reshape(1, h), wl3p, wr3p)
    p3 = _make_segsum(npad, ncp, cpad)(y3, ei)
    out = _mm_final(npad, cpad, r)(p3, dpart, z3, bl3p)
    return out[:n, :c]


# speedup vs baseline: 1.0027x; 1.0027x over previous
import functools
import jax
import jax.numpy as jnp
from jax import lax
from jax.experimental import pallas as pl
from jax.experimental.pallas import tpu as pltpu
from jax.experimental.pallas import tpu_sc as plsc

NC, NS, LANES = 2, 16, 16
CHUNK = 128


@functools.lru_cache(maxsize=None)
def _make_segsum(npad, n_chunks, w):
    nw = NC * NS
    cpw = n_chunks // nw
    G = 16
    ngroups = cpw // G
    rows_per_tile = npad // NS
    wb_chunks = rows_per_tile // CHUNK
    mesh = plsc.VectorSubcoreMesh(core_axis_name="c", subcore_axis_name="s")

    def body(y_hbm, ei_hbm, p_out, eig0, eig1, rows0, rows1, acc,
             sem0, sem1, gsem, esem0, esem1):
        ci = lax.axis_index("c")
        si = lax.axis_index("s")
        wid = si * NC + ci
        zeros16 = jnp.zeros((LANES,), jnp.float32)
        rows = (rows0, rows1)
        sems = (sem0, sem1)
        eigs = (eig0, eig1)
        esems = (esem0, esem1)

        @pl.loop(0, CHUNK)
        def _zr(r):
            for j in range(w // LANES):
                rows0[r, pl.ds(j * LANES, LANES)] = zeros16

        row0 = si * rows_per_tile

        @pl.loop(0, wb_chunks)
        def _za(k):
            pltpu.sync_copy(rows0, acc.at[pl.ds(row0 + k * CHUNK, CHUNK)])

        def eload(g, b):
            pltpu.async_copy(ei_hbm.at[pl.ds(wid * cpw + g * G, G)],
                             eigs[b], esems[b])

        def ewait(b):
            pltpu.make_async_copy(ei_hbm.at[pl.ds(0, G)], eigs[b],
                                  esems[b]).wait()

        eload(0, 0)
        plsc.subcore_barrier()

        def swait(b):
            pltpu.make_async_copy(rows[b], acc.at[eigs[0].at[0, 1]],
                                  sems[b]).wait()

        for g in range(ngroups):
            eb = g % 2
            ewait(eb)
            if g + 1 < ngroups:
                eload(g + 1, 1 - eb)
            eig = eigs[eb]

            def gath(k, b, eig=eig):
                pltpu.async_copy(y_hbm.at[eig.at[k, 0]], rows[b],
                                 gsem).wait()

            def sstart(k, b, eig=eig):
                pltpu.async_copy(rows[b], acc.at[eig.at[k, 1]], sems[b],
                                 add=True)

            if g == 0:
                gath(0, 0)
                sstart(0, 0)
                gath(1, 1)
                sstart(1, 1)
                k0 = 2
            else:
                k0 = 0

            @pl.loop(0, (G - k0) // 2)
            def _edges(k2):
                for b in (0, 1):
                    k = k0 + k2 * 2 + b
                    swait(b)
                    gath(k, b)
                    sstart(k, b)

        swait(0)
        swait(1)

        plsc.subcore_barrier()

        @pl.loop(0, wb_chunks)
        def _wb(k):
            r0 = row0 + k * CHUNK
            pltpu.sync_copy(acc.at[pl.ds(r0, CHUNK)], rows0)
            pltpu.sync_copy(rows0, p_out.at[ci, pl.ds(r0, CHUNK)])

    return pl.kernel(
        body,
        out_type=jax.ShapeDtypeStruct((NC, npad, w), jnp.float32),
        mesh=mesh,
        scratch_types=(
            pltpu.VMEM((G, 2, CHUNK), jnp.int32),
            pltpu.VMEM((G, 2, CHUNK), jnp.int32),
            pltpu.VMEM((CHUNK, w), jnp.float32),
            pltpu.VMEM((CHUNK, w), jnp.float32),
            pltpu.VMEM_SHARED((npad, w), jnp.float32),
            pltpu.SemaphoreType.DMA,
            pltpu.SemaphoreType.DMA,
            pltpu.SemaphoreType.DMA,
            pltpu.SemaphoreType.DMA,
            pltpu.SemaphoreType.DMA,
        ),
    )


@functools.lru_cache(maxsize=None)
def _make_deg(npad, n_chunks):
    w = 128
    nw = NC * NS
    cpw = n_chunks // nw
    rows_per_tile = npad // NS
    wb_chunks = rows_per_tile // CHUNK
    mesh = plsc.VectorSubcoreMesh(core_axis_name="c", subcore_axis_name="s")

    def body(ei_hbm, d_out, eis_v, rows_v, dacc, sem0, sem1):
        ci = lax.axis_index("c")
        si = lax.axis_index("s")
        wid = si * NC + ci
        zeros16 = jnp.zeros((LANES,), jnp.float32)
        sems = (sem0, sem1)

        @pl.loop(0, CHUNK)
        def _zr(r):
            for j in range(w // LANES):
                rows_v[r, pl.ds(j * LANES, LANES)] = zeros16

        row0 = si * rows_per_tile

        @pl.loop(0, wb_chunks)
        def _za(k):
            pltpu.sync_copy(rows_v, dacc.at[pl.ds(row0 + k * CHUNK, CHUNK)])

        ones16 = jnp.full((LANES,), 1.0, jnp.float32)

        @pl.loop(0, CHUNK)
        def _fo(r):
            rows_v[r, pl.ds(0, LANES)] = ones16

        pltpu.sync_copy(ei_hbm.at[pl.ds(wid * cpw, cpw)], eis_v)

        plsc.subcore_barrier()

        def start(k, b):
            pltpu.async_copy(rows_v, dacc.at[eis_v.at[k, 1]], sems[b],
                             add=True)

        def wait(b):
            pltpu.make_async_copy(rows_v, dacc.at[eis_v.at[0, 1]],
                                  sems[b]).wait()

        start(0, 0)

        @pl.loop(0, cpw // 2 - 1)
        def _edges(k2):
            for b in (0, 1):
                k = k2 * 2 + b
                start(k + 1, 1 - b)
                wait(b)

        start(cpw - 1, 1)
        wait(0)
        wait(1)

        plsc.subcore_barrier()

        @pl.loop(0, wb_chunks)
        def _wb(k):
            r0 = row0 + k * CHUNK
            pltpu.sync_copy(dacc.at[pl.ds(r0, CHUNK)], rows_v)
            pltpu.sync_copy(rows_v, d_out.at[ci, pl.ds(r0, CHUNK)])

    return pl.kernel(
        body,
        out_type=jax.ShapeDtypeStruct((NC, npad, w), jnp.float32),
        mesh=mesh,
        scratch_types=(
            pltpu.VMEM((cpw, 2, CHUNK), jnp.int32),
            pltpu.VMEM((CHUNK, w), jnp.float32),
            pltpu.VMEM_SHARED((npad, w), jnp.float32),
            pltpu.SemaphoreType.DMA,
            pltpu.SemaphoreType.DMA,
        ),
    )


def _mm_first(npad, d, h, r):
    def body(x_ref, wl_ref, wr_ref, y_ref, z_ref):
        xb = x_ref[...]
        y_ref[...] = jnp.dot(xb, wl_ref[...], preferred_element_type=jnp.float32)
        z_ref[...] = jnp.dot(xb, wr_ref[...], preferred_element_type=jnp.float32)

    return pl.pallas_call(
        body,
        grid=(npad // r,),
        in_specs=[
            pl.BlockSpec((r, d), lambda i: (i, 0)),
            pl.BlockSpec((d, h), lambda i: (0, 0)),
            pl.BlockSpec((d, h), lambda i: (0, 0)),
        ],
        out_specs=[
            pl.BlockSpec((r, h), lambda i: (i, 0)),
            pl.BlockSpec((r, h), lambda i: (i, 0)),
        ],
        out_shape=[
            jax.ShapeDtypeStruct((npad, h), jnp.float32),
            jax.ShapeDtypeStruct((npad, h), jnp.float32),
        ],
    )


def _mm_combine(npad, w_in, w_out, r):
    def body(p_ref, d_ref, z_ref, b_ref, wl_ref, wr_ref, y_ref, z2_ref):
        s = p_ref[0] + p_ref[1]
        deg = d_ref[0, :, 0] + d_ref[1, :, 0]
        dc = jnp.maximum(deg, 1.0)
        hh = s / dc[:, None] + b_ref[...] + z_ref[...]
        hh = jnp.maximum(hh, 0.0)
        y_ref[...] = jnp.dot(hh, wl_ref[...], preferred_element_type=jnp.float32)
        z2_ref[...] = jnp.dot(hh, wr_ref[...], preferred_element_type=jnp.float32)

    return pl.pallas_call(
        body,
        grid=(npad // r,),
        in_specs=[
            pl.BlockSpec((NC, r, w_in), lambda i: (0, i, 0)),
            pl.BlockSpec((NC, r, 128), lambda i: (0, i, 0)),
            pl.BlockSpec((r, w_in), lambda i: (i, 0)),
            pl.BlockSpec((1, w_in), lambda i: (0, 0)),
            pl.BlockSpec((w_in, w_out), lambda i: (0, 0)),
            pl.BlockSpec((w_in, w_out), lambda i: (0, 0)),
        ],
        out_specs=[
            pl.BlockSpec((r, w_out), lambda i: (i, 0)),
            pl.BlockSpec((r, w_out), lambda i: (i, 0)),
        ],
        out_shape=[
            jax.ShapeDtypeStruct((npad, w_out), jnp.float32),
            jax.ShapeDtypeStruct((npad, w_out), jnp.float32),
        ],
    )


def _mm_final(npad, w, r):
    def body(p_ref, d_ref, z_ref, b_ref, o_ref):
        s = p_ref[0] + p_ref[1]
        deg = d_ref[0, :, 0] + d_ref[1, :, 0]
        dc = jnp.maximum(deg, 1.0)
        o_ref[...] = s / dc[:, None] + b_ref[...] + z_ref[...]

    return pl.pallas_call(
        body,
        grid=(npad // r,),
        in_specs=[
            pl.BlockSpec((NC, r, w), lambda i: (0, i, 0)),
            pl.BlockSpec((NC, r, 128), lambda i: (0, i, 0)),
            pl.BlockSpec((r, w), lambda i: (i, 0)),
            pl.BlockSpec((1, w), lambda i: (0, 0)),
        ],
        out_specs=pl.BlockSpec((r, w), lambda i: (i, 0)),
        out_shape=jax.ShapeDtypeStruct((npad, w), jnp.float32),
    )


def kernel(x, edge_index, relations, Wl1, bl1, Wr1, Wl2, bl2, Wr2, Wl3, bl3, Wr3):
    n, d = x.shape
    e = edge_index.shape[1]
    h = Wl1.shape[1]
    c = Wl3.shape[1]
    cpad = 128
    tile = NS * CHUNK
    npad = ((n + tile - 1) // tile) * tile
    if npad == n:
        npad += tile
    r = 1024

    x_pad = jnp.pad(x, ((0, npad - n), (0, 0)))
    nchunks = e // CHUNK
    nw = NC * NS
    ncp = -(-nchunks // (16 * nw)) * (16 * nw)
    ei = edge_index.astype(jnp.int32).reshape(2, nchunks, CHUNK)
    ei = jnp.transpose(ei, (1, 0, 2))
    if ncp > nchunks:
        npc = ncp - nchunks
        spread = n + (jnp.arange(npc * CHUNK, dtype=jnp.int32) % (npad - n))
        pad = jnp.stack(
            [jnp.zeros((npc, CHUNK), jnp.int32),
             spread.reshape(npc, CHUNK)], axis=1)
        ei = jnp.concatenate([ei, pad], axis=0)
    wl3p = jnp.pad(Wl3, ((0, 0), (0, cpad - c)))
    wr3p = jnp.pad(Wr3, ((0, 0), (0, cpad - c)))
    bl3p = jnp.pad(bl3, (0, cpad - c)).reshape(1, cpad)

    dpart = _make_deg(npad, ncp)(ei)
    y1, z1 = _mm_first(npad, d, h, r)(x_pad, Wl1, Wr1)
    p1 = _make_segsum(npad, ncp, h)(y1, ei)
    y2, z2 = _mm_combine(npad, h, h, r)(p1, dpart, z1, bl1.reshape(1, h), Wl2, Wr2)
    p2 = _make_segsum(npad, ncp, h)(y2, ei)
    y3, z3 = _mm_combine(npad, h, cpad, r)(p2, dpart, z2, bl2.reshape(1, h), wl3p, wr3p)
    p3 = _make_segsum(npad, ncp, cpad)(y3, ei)
    out = _mm_final(npad, cpad, r)(p3, dpart, z3, bl3p)
    return out[:n, :c]


# speedup vs baseline: 3.0173x; 3.0091x over previous
import functools
import jax
import jax.numpy as jnp
from jax import lax
from jax.experimental import pallas as pl
from jax.experimental.pallas import tpu as pltpu
from jax.experimental.pallas import tpu_sc as plsc

NC, NS, LANES = 2, 16, 16
CHUNK = 128


@functools.lru_cache(maxsize=None)
def _make_segsum(npad, n_chunks, w):
    nw = NC * NS
    cpw = n_chunks // nw
    G = 16
    ngroups = cpw // G
    rows_per_tile = npad // NS
    wb_chunks = rows_per_tile // CHUNK
    mesh = plsc.VectorSubcoreMesh(core_axis_name="c", subcore_axis_name="s")

    def body(y_hbm, ei_hbm, p_out, eig0, eig1, rows0, rows1, acc,
             sem0, sem1, gsem, esem0, esem1):
        ci = lax.axis_index("c")
        si = lax.axis_index("s")
        wid = si * NC + ci
        zeros16 = jnp.zeros((LANES,), jnp.float32)
        rows = (rows0, rows1)
        sems = (sem0, sem1)
        eigs = (eig0, eig1)
        esems = (esem0, esem1)

        @pl.loop(0, CHUNK)
        def _zr(r):
            for j in range(w // LANES):
                rows0[r, pl.ds(j * LANES, LANES)] = zeros16

        row0 = si * rows_per_tile

        @pl.loop(0, wb_chunks)
        def _za(k):
            pltpu.sync_copy(rows0, acc.at[pl.ds(row0 + k * CHUNK, CHUNK)])

        def eload(g, b):
            pltpu.async_copy(ei_hbm.at[pl.ds(wid * cpw + g * G, G)],
                             eigs[b], esems[b])

        def ewait(b):
            pltpu.make_async_copy(ei_hbm.at[pl.ds(0, G)], eigs[b],
                                  esems[b]).wait()

        eload(0, 0)
        plsc.subcore_barrier()

        def swait(b):
            pltpu.make_async_copy(rows[b], acc.at[eigs[0].at[0, 1]],
                                  sems[b]).wait()

        for g in range(ngroups):
            eb = g % 2
            ewait(eb)
            if g + 1 < ngroups:
                eload(g + 1, 1 - eb)
            eig = eigs[eb]

            def gath(k, b, eig=eig):
                pltpu.async_copy(y_hbm.at[eig.at[k, 0]], rows[b],
                                 gsem).wait()

            def sstart(k, b, eig=eig):
                pltpu.async_copy(rows[b], acc.at[eig.at[k, 1]], sems[b],
                                 add=True)

            if g == 0:
                gath(0, 0)
                sstart(0, 0)
                gath(1, 1)
                sstart(1, 1)
                k0 = 2
            else:
                k0 = 0

            @pl.loop(0, (G - k0) // 2)
            def _edges(k2):
                for b in (0, 1):
                    k = k0 + k2 * 2 + b
                    swait(b)
                    gath(k, b)
                    sstart(k, b)

        swait(0)
        swait(1)

        plsc.subcore_barrier()

        @pl.loop(0, wb_chunks)
        def _wb(k):
            r0 = row0 + k * CHUNK
            pltpu.sync_copy(acc.at[pl.ds(r0, CHUNK)], rows0)
            pltpu.sync_copy(rows0, p_out.at[ci, pl.ds(r0, CHUNK)])

    return pl.kernel(
        body,
        out_type=jax.ShapeDtypeStruct((NC, npad, w), jnp.float32),
        mesh=mesh,
        scratch_types=(
            pltpu.VMEM((G, 2, CHUNK), jnp.int32),
            pltpu.VMEM((G, 2, CHUNK), jnp.int32),
            pltpu.VMEM((CHUNK, w), jnp.float32),
            pltpu.VMEM((CHUNK, w), jnp.float32),
            pltpu.VMEM_SHARED((npad, w), jnp.float32),
            pltpu.SemaphoreType.DMA,
            pltpu.SemaphoreType.DMA,
            pltpu.SemaphoreType.DMA,
            pltpu.SemaphoreType.DMA,
            pltpu.SemaphoreType.DMA,
        ),
    )


@functools.lru_cache(maxsize=None)
def _make_deg(npad, n_chunks):
    w = 128
    nw = NC * NS
    cpw = n_chunks // nw
    rows_per_tile = npad // NS
    wb_chunks = rows_per_tile // CHUNK
    mesh = plsc.VectorSubcoreMesh(core_axis_name="c", subcore_axis_name="s")

    def body(ei_hbm, d_out, eis_v, rows_v, dacc, sem0, sem1):
        ci = lax.axis_index("c")
        si = lax.axis_index("s")
        wid = si * NC + ci
        zeros16 = jnp.zeros((LANES,), jnp.float32)
        sems = (sem0, sem1)

        @pl.loop(0, CHUNK)
        def _zr(r):
            for j in range(w // LANES):
                rows_v[r, pl.ds(j * LANES, LANES)] = zeros16

        row0 = si * rows_per_tile

        @pl.loop(0, wb_chunks)
        def _za(k):
            pltpu.sync_copy(rows_v, dacc.at[pl.ds(row0 + k * CHUNK, CHUNK)])

        ones16 = jnp.full((LANES,), 1.0, jnp.float32)

        @pl.loop(0, CHUNK)
        def _fo(r):
            rows_v[r, pl.ds(0, LANES)] = ones16

        pltpu.sync_copy(ei_hbm.at[pl.ds(wid * cpw, cpw)], eis_v)

        plsc.subcore_barrier()

        def start(k, b):
            pltpu.async_copy(rows_v, dacc.at[eis_v.at[k, 1]], sems[b],
                             add=True)

        def wait(b):
            pltpu.make_async_copy(rows_v, dacc.at[eis_v.at[0, 1]],
                                  sems[b]).wait()

        start(0, 0)

        @pl.loop(0, cpw // 2 - 1)
        def _edges(k2):
            for b in (0, 1):
                k = k2 * 2 + b
                start(k + 1, 1 - b)
                wait(b)

        start(cpw - 1, 1)
        wait(0)
        wait(1)

        plsc.subcore_barrier()

        @pl.loop(0, wb_chunks)
        def _wb(k):
            r0 = row0 + k * CHUNK
            pltpu.sync_copy(dacc.at[pl.ds(r0, CHUNK)], rows_v)
            pltpu.sync_copy(rows_v, d_out.at[ci, pl.ds(r0, CHUNK)])

    return pl.kernel(
        body,
        out_type=jax.ShapeDtypeStruct((NC, npad, w), jnp.float32),
        mesh=mesh,
        scratch_types=(
            pltpu.VMEM((cpw, 2, CHUNK), jnp.int32),
            pltpu.VMEM((CHUNK, w), jnp.float32),
            pltpu.VMEM_SHARED((npad, w), jnp.float32),
            pltpu.SemaphoreType.DMA,
            pltpu.SemaphoreType.DMA,
        ),
    )


def _mm_first(npad, d, h, r):
    def body(x_ref, wl_ref, wr_ref, y_ref, z_ref):
        xb = x_ref[...]
        y_ref[...] = jnp.dot(xb, wl_ref[...], preferred_element_type=jnp.float32)
        z_ref[...] = jnp.dot(xb, wr_ref[...], preferred_element_type=jnp.float32)

    return pl.pallas_call(
        body,
        grid=(npad // r,),
        in_specs=[
            pl.BlockSpec((r, d), lambda i: (i, 0)),
            pl.BlockSpec((d, h), lambda i: (0, 0)),
            pl.BlockSpec((d, h), lambda i: (0, 0)),
        ],
        out_specs=[
            pl.BlockSpec((r, h), lambda i: (i, 0)),
            pl.BlockSpec((r, h), lambda i: (i, 0)),
        ],
        out_shape=[
            jax.ShapeDtypeStruct((npad, h), jnp.float32),
            jax.ShapeDtypeStruct((npad, h), jnp.float32),
        ],
    )


def _mm_combine(npad, w_in, w_out, r):
    def body(p_ref, d_ref, z_ref, b_ref, wl_ref, wr_ref, y_ref, z2_ref):
        s = p_ref[0] + p_ref[1]
        deg = d_ref[0, :, 0] + d_ref[1, :, 0]
        dc = jnp.maximum(deg, 1.0)
        hh = s / dc[:, None] + b_ref[...] + z_ref[...]
        hh = jnp.maximum(hh, 0.0)
        y_ref[...] = jnp.dot(hh, wl_ref[...], preferred_element_type=jnp.float32)
        z2_ref[...] = jnp.dot(hh, wr_ref[...], preferred_element_type=jnp.float32)

    return pl.pallas_call(
        body,
        grid=(npad // r,),
        in_specs=[
            pl.BlockSpec((NC, r, w_in), lambda i: (0, i, 0)),
            pl.BlockSpec((NC, r, 128), lambda i: (0, i, 0)),
            pl.BlockSpec((r, w_in), lambda i: (i, 0)),
            pl.BlockSpec((1, w_in), lambda i: (0, 0)),
            pl.BlockSpec((w_in, w_out), lambda i: (0, 0)),
            pl.BlockSpec((w_in, w_out), lambda i: (0, 0)),
        ],
        out_specs=[
            pl.BlockSpec((r, w_out), lambda i: (i, 0)),
            pl.BlockSpec((r, w_out), lambda i: (i, 0)),
        ],
        out_shape=[
            jax.ShapeDtypeStruct((npad, w_out), jnp.float32),
            jax.ShapeDtypeStruct((npad, w_out), jnp.float32),
        ],
    )


def _mm_final(npad, w, r):
    def body(p_ref, d_ref, z_ref, b_ref, o_ref):
        s = p_ref[0] + p_ref[1]
        deg = d_ref[0, :, 0] + d_ref[1, :, 0]
        dc = jnp.maximum(deg, 1.0)
        o_ref[...] = s / dc[:, None] + b_ref[...] + z_ref[...]

    return pl.pallas_call(
        body,
        grid=(npad // r,),
        in_specs=[
            pl.BlockSpec((NC, r, w), lambda i: (0, i, 0)),
            pl.BlockSpec((NC, r, 128), lambda i: (0, i, 0)),
            pl.BlockSpec((r, w), lambda i: (i, 0)),
            pl.BlockSpec((1, w), lambda i: (0, 0)),
        ],
        out_specs=pl.BlockSpec((r, w), lambda i: (i, 0)),
        out_shape=jax.ShapeDtypeStruct((npad, w), jnp.float32),
    )


def kernel(x, edge_index, relations, Wl1, bl1, Wr1, Wl2, bl2, Wr2, Wl3, bl3, Wr3):
    n, d = x.shape
    e = edge_index.shape[1]
    h = Wl1.shape[1]
    c = Wl3.shape[1]
    cpad = 128
    tile = NS * CHUNK
    npad = ((n + tile - 1) // tile) * tile
    if npad == n:
        npad += tile
    r = 1024

    x_pad = jnp.pad(x, ((0, npad - n), (0, 0)))
    nchunks = e // CHUNK
    nw = NC * NS
    ncp = -(-nchunks // (16 * nw)) * (16 * nw)
    ei = edge_index.astype(jnp.int32).reshape(2, nchunks, CHUNK)
    ei = jnp.transpose(ei, (1, 0, 2))
    if ncp > nchunks:
        npc = ncp - nchunks
        ar = jnp.arange(npc * CHUNK, dtype=jnp.int32)
        pad = jnp.stack(
            [(ar % n).reshape(npc, CHUNK),
             (n + ar % (npad - n)).reshape(npc, CHUNK)], axis=1)
        ei = jnp.concatenate([ei, pad], axis=0)
    wl3p = jnp.pad(Wl3, ((0, 0), (0, cpad - c)))
    wr3p = jnp.pad(Wr3, ((0, 0), (0, cpad - c)))
    bl3p = jnp.pad(bl3, (0, cpad - c)).reshape(1, cpad)

    dpart = _make_deg(npad, ncp)(ei)
    y1, z1 = _mm_first(npad, d, h, r)(x_pad, Wl1, Wr1)
    p1 = _make_segsum(npad, ncp, h)(y1, ei)
    y2, z2 = _mm_combine(npad, h, h, r)(p1, dpart, z1, bl1.reshape(1, h), Wl2, Wr2)
    p2 = _make_segsum(npad, ncp, h)(y2, ei)
    y3, z3 = _mm_combine(npad, h, cpad, r)(p2, dpart, z2, bl2.reshape(1, h), wl3p, wr3p)
    p3 = _make_segsum(npad, ncp, cpad)(y3, ei)
    out = _mm_final(npad, cpad, r)(p3, dpart, z3, bl3p)
    return out[:n, :c]


# speedup vs baseline: 3.2045x; 1.0621x over previous
import functools
import jax
import jax.numpy as jnp
from jax import lax
from jax.experimental import pallas as pl
from jax.experimental.pallas import tpu as pltpu
from jax.experimental.pallas import tpu_sc as plsc

NC, NS, LANES = 2, 16, 16
CHUNK = 128


@functools.lru_cache(maxsize=None)
def _make_segsum(npad, n_chunks, w):
    nw = NC * NS
    cpw = n_chunks // nw
    G = 16
    ngroups = cpw // G
    rows_per_tile = npad // NS
    wb_chunks = rows_per_tile // CHUNK
    mesh = plsc.VectorSubcoreMesh(core_axis_name="c", subcore_axis_name="s")

    def body(y_hbm, ei_hbm, p_out, eig0, eig1, rows0, rows1, acc,
             sem0, sem1, gsem, esem0, esem1):
        ci = lax.axis_index("c")
        si = lax.axis_index("s")
        wid = si * NC + ci
        zeros16 = jnp.zeros((LANES,), jnp.float32)
        rows = (rows0, rows1)
        sems = (sem0, sem1)
        eigs = (eig0, eig1)
        esems = (esem0, esem1)

        @pl.loop(0, CHUNK)
        def _zr(r):
            for j in range(w // LANES):
                rows0[r, pl.ds(j * LANES, LANES)] = zeros16

        row0 = si * rows_per_tile

        @pl.loop(0, wb_chunks)
        def _za(k):
            pltpu.sync_copy(rows0, acc.at[pl.ds(row0 + k * CHUNK, CHUNK)])

        def eload(g, b):
            pltpu.async_copy(ei_hbm.at[pl.ds(wid * cpw + g * G, G)],
                             eigs[b], esems[b])

        def ewait(b):
            pltpu.make_async_copy(ei_hbm.at[pl.ds(0, G)], eigs[b],
                                  esems[b]).wait()

        eload(0, 0)
        plsc.subcore_barrier()

        def swait(b):
            pltpu.make_async_copy(rows[b], acc.at[eigs[0].at[0, 1]],
                                  sems[b]).wait()

        for g in range(ngroups):
            eb = g % 2
            ewait(eb)
            if g + 1 < ngroups:
                eload(g + 1, 1 - eb)
            eig = eigs[eb]

            def gath(k, b, eig=eig):
                pltpu.async_copy(y_hbm.at[eig.at[k, 0]], rows[b],
                                 gsem).wait()

            def sstart(k, b, eig=eig):
                pltpu.async_copy(rows[b], acc.at[eig.at[k, 1]], sems[b],
                                 add=True)

            if g == 0:
                gath(0, 0)
                sstart(0, 0)
                gath(1, 1)
                sstart(1, 1)
                k0 = 2
            else:
                k0 = 0

            @pl.loop(0, (G - k0) // 2)
            def _edges(k2):
                for b in (0, 1):
                    k = k0 + k2 * 2 + b
                    swait(b)
                    gath(k, b)
                    sstart(k, b)

        swait(0)
        swait(1)

        plsc.subcore_barrier()

        @pl.loop(0, wb_chunks)
        def _wb(k):
            r0 = row0 + k * CHUNK
            pltpu.sync_copy(acc.at[pl.ds(r0, CHUNK)], rows0)
            pltpu.sync_copy(rows0, p_out.at[ci, pl.ds(r0, CHUNK)])

    params = None
    if w % 128 != 0:
        params = pltpu.CompilerParams(use_tc_tiling_on_sc=False)
    return pl.kernel(
        body,
        out_type=jax.ShapeDtypeStruct((NC, npad, w), jnp.float32),
        mesh=mesh,
        compiler_params=params,
        scratch_types=(
            pltpu.VMEM((G, 2, CHUNK), jnp.int32),
            pltpu.VMEM((G, 2, CHUNK), jnp.int32),
            pltpu.VMEM((CHUNK, w), jnp.float32),
            pltpu.VMEM((CHUNK, w), jnp.float32),
            pltpu.VMEM_SHARED((npad, w), jnp.float32),
            pltpu.SemaphoreType.DMA,
            pltpu.SemaphoreType.DMA,
            pltpu.SemaphoreType.DMA,
            pltpu.SemaphoreType.DMA,
            pltpu.SemaphoreType.DMA,
        ),
    )


@functools.lru_cache(maxsize=None)
def _make_deg(npad, n_chunks):
    w = 128
    nw = NC * NS
    cpw = n_chunks // nw
    rows_per_tile = npad // NS
    wb_chunks = rows_per_tile // CHUNK
    mesh = plsc.VectorSubcoreMesh(core_axis_name="c", subcore_axis_name="s")

    def body(ei_hbm, d_out, eis_v, rows_v, dacc, sem0, sem1):
        ci = lax.axis_index("c")
        si = lax.axis_index("s")
        wid = si * NC + ci
        zeros16 = jnp.zeros((LANES,), jnp.float32)
        sems = (sem0, sem1)

        @pl.loop(0, CHUNK)
        def _zr(r):
            for j in range(w // LANES):
                rows_v[r, pl.ds(j * LANES, LANES)] = zeros16

        row0 = si * rows_per_tile

        @pl.loop(0, wb_chunks)
        def _za(k):
            pltpu.sync_copy(rows_v, dacc.at[pl.ds(row0 + k * CHUNK, CHUNK)])

        ones16 = jnp.full((LANES,), 1.0, jnp.float32)

        @pl.loop(0, CHUNK)
        def _fo(r):
            rows_v[r, pl.ds(0, LANES)] = ones16

        pltpu.sync_copy(ei_hbm.at[pl.ds(wid * cpw, cpw)], eis_v)

        plsc.subcore_barrier()

        def start(k, b):
            pltpu.async_copy(rows_v, dacc.at[eis_v.at[k, 1]], sems[b],
                             add=True)

        def wait(b):
            pltpu.make_async_copy(rows_v, dacc.at[eis_v.at[0, 1]],
                                  sems[b]).wait()

        start(0, 0)

        @pl.loop(0, cpw // 2 - 1)
        def _edges(k2):
            for b in (0, 1):
                k = k2 * 2 + b
                start(k + 1, 1 - b)
                wait(b)

        start(cpw - 1, 1)
        wait(0)
        wait(1)

        plsc.subcore_barrier()

        @pl.loop(0, wb_chunks)
        def _wb(k):
            r0 = row0 + k * CHUNK
            pltpu.sync_copy(dacc.at[pl.ds(r0, CHUNK)], rows_v)
            pltpu.sync_copy(rows_v, d_out.at[ci, pl.ds(r0, CHUNK)])

    return pl.kernel(
        body,
        out_type=jax.ShapeDtypeStruct((NC, npad, w), jnp.float32),
        mesh=mesh,
        scratch_types=(
            pltpu.VMEM((cpw, 2, CHUNK), jnp.int32),
            pltpu.VMEM((CHUNK, w), jnp.float32),
            pltpu.VMEM_SHARED((npad, w), jnp.float32),
            pltpu.SemaphoreType.DMA,
            pltpu.SemaphoreType.DMA,
        ),
    )


def _mm_first(npad, d, h, r):
    def body(x_ref, wl_ref, wr_ref, y_ref, z_ref):
        xb = x_ref[...]
        y_ref[...] = jnp.dot(xb, wl_ref[...], preferred_element_type=jnp.float32)
        z_ref[...] = jnp.dot(xb, wr_ref[...], preferred_element_type=jnp.float32)

    return pl.pallas_call(
        body,
        grid=(npad // r,),
        in_specs=[
            pl.BlockSpec((r, d), lambda i: (i, 0)),
            pl.BlockSpec((d, h), lambda i: (0, 0)),
            pl.BlockSpec((d, h), lambda i: (0, 0)),
        ],
        out_specs=[
            pl.BlockSpec((r, h), lambda i: (i, 0)),
            pl.BlockSpec((r, h), lambda i: (i, 0)),
        ],
        out_shape=[
            jax.ShapeDtypeStruct((npad, h), jnp.float32),
            jax.ShapeDtypeStruct((npad, h), jnp.float32),
        ],
    )


def _mm_combine(npad, w_in, w_out, r):
    def body(p_ref, d_ref, z_ref, b_ref, wl_ref, wr_ref, y_ref, z2_ref):
        s = p_ref[0] + p_ref[1]
        deg = d_ref[0, :, 0] + d_ref[1, :, 0]
        dc = jnp.maximum(deg, 1.0)
        hh = s / dc[:, None] + b_ref[...] + z_ref[...]
        hh = jnp.maximum(hh, 0.0)
        y_ref[...] = jnp.dot(hh, wl_ref[...], preferred_element_type=jnp.float32)
        z2_ref[...] = jnp.dot(hh, wr_ref[...], preferred_element_type=jnp.float32)

    return pl.pallas_call(
        body,
        grid=(npad // r,),
        in_specs=[
            pl.BlockSpec((NC, r, w_in), lambda i: (0, i, 0)),
            pl.BlockSpec((NC, r, 128), lambda i: (0, i, 0)),
            pl.BlockSpec((r, w_in), lambda i: (i, 0)),
            pl.BlockSpec((1, w_in), lambda i: (0, 0)),
            pl.BlockSpec((w_in, w_out), lambda i: (0, 0)),
            pl.BlockSpec((w_in, w_out), lambda i: (0, 0)),
        ],
        out_specs=[
            pl.BlockSpec((r, w_out), lambda i: (i, 0)),
            pl.BlockSpec((r, w_out), lambda i: (i, 0)),
        ],
        out_shape=[
            jax.ShapeDtypeStruct((npad, w_out), jnp.float32),
            jax.ShapeDtypeStruct((npad, w_out), jnp.float32),
        ],
    )


def _mm_final(npad, w, r):
    def body(p_ref, d_ref, z_ref, b_ref, o_ref):
        s = p_ref[0] + p_ref[1]
        deg = d_ref[0, :, 0] + d_ref[1, :, 0]
        dc = jnp.maximum(deg, 1.0)
        o_ref[...] = s / dc[:, None] + b_ref[...] + z_ref[...]

    return pl.pallas_call(
        body,
        grid=(npad // r,),
        in_specs=[
            pl.BlockSpec((NC, r, w), lambda i: (0, i, 0)),
            pl.BlockSpec((NC, r, 128), lambda i: (0, i, 0)),
            pl.BlockSpec((r, w), lambda i: (i, 0)),
            pl.BlockSpec((1, w), lambda i: (0, 0)),
        ],
        out_specs=pl.BlockSpec((r, w), lambda i: (i, 0)),
        out_shape=jax.ShapeDtypeStruct((npad, w), jnp.float32),
    )


def kernel(x, edge_index, relations, Wl1, bl1, Wr1, Wl2, bl2, Wr2, Wl3, bl3, Wr3):
    n, d = x.shape
    e = edge_index.shape[1]
    h = Wl1.shape[1]
    c = Wl3.shape[1]
    cpad = 48
    tile = NS * CHUNK
    npad = ((n + tile - 1) // tile) * tile
    if npad == n:
        npad += tile
    r = 1024

    x_pad = jnp.pad(x, ((0, npad - n), (0, 0)))
    nchunks = e // CHUNK
    nw = NC * NS
    ncp = -(-nchunks // (16 * nw)) * (16 * nw)
    ei = edge_index.astype(jnp.int32).reshape(2, nchunks, CHUNK)
    ei = jnp.transpose(ei, (1, 0, 2))
    if ncp > nchunks:
        npc = ncp - nchunks
        ar = jnp.arange(npc * CHUNK, dtype=jnp.int32)
        pad = jnp.stack(
            [(ar % n).reshape(npc, CHUNK),
             (n + ar % (npad - n)).reshape(npc, CHUNK)], axis=1)
        ei = jnp.concatenate([ei, pad], axis=0)
    wl3p = jnp.pad(Wl3, ((0, 0), (0, cpad - c)))
    wr3p = jnp.pad(Wr3, ((0, 0), (0, cpad - c)))
    bl3p = jnp.pad(bl3, (0, cpad - c)).reshape(1, cpad)

    dpart = _make_deg(npad, ncp)(ei)
    y1, z1 = _mm_first(npad, d, h, r)(x_pad, Wl1, Wr1)
    p1 = _make_segsum(npad, ncp, h)(y1, ei)
    y2, z2 = _mm_combine(npad, h, h, r)(p1, dpart, z1, bl1.reshape(1, h), Wl2, Wr2)
    p2 = _make_segsum(npad, ncp, h)(y2, ei)
    y3, z3 = _mm_combine(npad, h, cpad, r)(p2, dpart, z2, bl2.reshape(1, h), wl3p, wr3p)
    p3 = _make_segsum(npad, ncp, cpad)(y3, ei)
    out = _mm_final(npad, cpad, r)(p3, dpart, z3, bl3p)
    return out[:n, :c]


# speedup vs baseline: 3.5655x; 1.1127x over previous
import functools
import jax
import jax.numpy as jnp
from jax import lax
from jax.experimental import pallas as pl
from jax.experimental.pallas import tpu as pltpu
from jax.experimental.pallas import tpu_sc as plsc

NC, NS, LANES = 2, 16, 16
CHUNK = 128


@functools.lru_cache(maxsize=None)
def _make_segsum(npad, n_chunks, w):
    nw = NC * NS
    cpw = n_chunks // nw
    G = 16
    ngroups = cpw // G
    rows_per_tile = npad // NS
    wb_chunks = rows_per_tile // CHUNK
    mesh = plsc.VectorSubcoreMesh(core_axis_name="c", subcore_axis_name="s")

    def body(y_hbm, ei_hbm, p_out, eig0, eig1, rows0, rows1, acc,
             sem0, sem1, gsem, esem0, esem1):
        ci = lax.axis_index("c")
        si = lax.axis_index("s")
        wid = si * NC + ci
        zeros16 = jnp.zeros((LANES,), jnp.float32)
        rows = (rows0, rows1)
        sems = (sem0, sem1)
        eigs = (eig0, eig1)
        esems = (esem0, esem1)

        @pl.loop(0, CHUNK)
        def _zr(r):
            for j in range(w // LANES):
                rows0[r, pl.ds(j * LANES, LANES)] = zeros16

        row0 = si * rows_per_tile

        @pl.loop(0, wb_chunks)
        def _za(k):
            pltpu.sync_copy(rows0, acc.at[pl.ds(row0 + k * CHUNK, CHUNK)])

        def eload(g, b):
            pltpu.async_copy(ei_hbm.at[pl.ds(wid * cpw + g * G, G)],
                             eigs[b], esems[b])

        def ewait(b):
            pltpu.make_async_copy(ei_hbm.at[pl.ds(0, G)], eigs[b],
                                  esems[b]).wait()

        eload(0, 0)
        plsc.subcore_barrier()

        def swait(b):
            pltpu.make_async_copy(rows[b], acc.at[eigs[0].at[0, 1]],
                                  sems[b]).wait()

        for g in range(ngroups):
            eb = g % 2
            ewait(eb)
            if g + 1 < ngroups:
                eload(g + 1, 1 - eb)
            eig = eigs[eb]

            def gath(k, b, eig=eig):
                pltpu.async_copy(y_hbm.at[eig.at[k, 0]], rows[b],
                                 gsem).wait()

            def sstart(k, b, eig=eig):
                pltpu.async_copy(rows[b], acc.at[eig.at[k, 1]], sems[b],
                                 add=True)

            if g == 0:
                gath(0, 0)
                sstart(0, 0)
                gath(1, 1)
                sstart(1, 1)
                k0 = 2
            else:
                k0 = 0

            @pl.loop(0, (G - k0) // 2)
            def _edges(k2):
                for b in (0, 1):
                    k = k0 + k2 * 2 + b
                    swait(b)
                    gath(k, b)
                    sstart(k, b)

        swait(0)
        swait(1)

        plsc.subcore_barrier()

        @pl.loop(0, wb_chunks)
        def _wb(k):
            r0 = row0 + k * CHUNK
            pltpu.sync_copy(acc.at[pl.ds(r0, CHUNK)], rows0)
            pltpu.sync_copy(rows0, p_out.at[ci, pl.ds(r0, CHUNK)])

    params = None
    if w % 128 != 0:
        params = pltpu.CompilerParams(use_tc_tiling_on_sc=False)
    return pl.kernel(
        body,
        out_type=jax.ShapeDtypeStruct((NC, npad, w), jnp.float32),
        mesh=mesh,
        compiler_params=params,
        scratch_types=(
            pltpu.VMEM((G, 2, CHUNK), jnp.int32),
            pltpu.VMEM((G, 2, CHUNK), jnp.int32),
            pltpu.VMEM((CHUNK, w), jnp.float32),
            pltpu.VMEM((CHUNK, w), jnp.float32),
            pltpu.VMEM_SHARED((npad, w), jnp.float32),
            pltpu.SemaphoreType.DMA,
            pltpu.SemaphoreType.DMA,
            pltpu.SemaphoreType.DMA,
            pltpu.SemaphoreType.DMA,
            pltpu.SemaphoreType.DMA,
        ),
    )


@functools.lru_cache(maxsize=None)
def _make_deg(npad, n_chunks):
    w = LANES
    nw = NC * NS
    cpw = n_chunks // nw
    rows_per_tile = npad // NS
    wb_chunks = rows_per_tile // CHUNK
    mesh = plsc.VectorSubcoreMesh(core_axis_name="c", subcore_axis_name="s")

    def body(ei_hbm, d_out, eis_v, rows_v, dacc, sem0, sem1):
        ci = lax.axis_index("c")
        si = lax.axis_index("s")
        wid = si * NC + ci
        zeros16 = jnp.zeros((LANES,), jnp.float32)
        sems = (sem0, sem1)

        @pl.loop(0, CHUNK)
        def _zr(r):
            for j in range(w // LANES):
                rows_v[r, pl.ds(j * LANES, LANES)] = zeros16

        row0 = si * rows_per_tile

        @pl.loop(0, wb_chunks)
        def _za(k):
            pltpu.sync_copy(rows_v, dacc.at[pl.ds(row0 + k * CHUNK, CHUNK)])

        ones16 = jnp.full((LANES,), 1.0, jnp.float32)

        @pl.loop(0, CHUNK)
        def _fo(r):
            rows_v[r, pl.ds(0, LANES)] = ones16

        pltpu.sync_copy(ei_hbm.at[pl.ds(wid * cpw, cpw)], eis_v)

        plsc.subcore_barrier()

        def start(k, b):
            pltpu.async_copy(rows_v, dacc.at[eis_v.at[k, 1]], sems[b],
                             add=True)

        def wait(b):
            pltpu.make_async_copy(rows_v, dacc.at[eis_v.at[0, 1]],
                                  sems[b]).wait()

        start(0, 0)

        @pl.loop(0, cpw // 2 - 1)
        def _edges(k2):
            for b in (0, 1):
                k = k2 * 2 + b
                start(k + 1, 1 - b)
                wait(b)

        start(cpw - 1, 1)
        wait(0)
        wait(1)

        plsc.subcore_barrier()

        @pl.loop(0, wb_chunks)
        def _wb(k):
            r0 = row0 + k * CHUNK
            pltpu.sync_copy(dacc.at[pl.ds(r0, CHUNK)], rows_v)
            pltpu.sync_copy(rows_v, d_out.at[ci, pl.ds(r0, CHUNK)])

    return pl.kernel(
        body,
        out_type=jax.ShapeDtypeStruct((NC, npad, w), jnp.float32),
        mesh=mesh,
        compiler_params=pltpu.CompilerParams(use_tc_tiling_on_sc=False),
        scratch_types=(
            pltpu.VMEM((cpw, 2, CHUNK), jnp.int32),
            pltpu.VMEM((CHUNK, w), jnp.float32),
            pltpu.VMEM_SHARED((npad, w), jnp.float32),
            pltpu.SemaphoreType.DMA,
            pltpu.SemaphoreType.DMA,
        ),
    )


def _mm_first(npad, d, h, r):
    def body(x_ref, wl_ref, wr_ref, y_ref, z_ref):
        xb = x_ref[...]
        y_ref[...] = jnp.dot(xb, wl_ref[...], preferred_element_type=jnp.float32)
        z_ref[...] = jnp.dot(xb, wr_ref[...], preferred_element_type=jnp.float32)

    return pl.pallas_call(
        body,
        grid=(npad // r,),
        in_specs=[
            pl.BlockSpec((r, d), lambda i: (i, 0)),
            pl.BlockSpec((d, h), lambda i: (0, 0)),
            pl.BlockSpec((d, h), lambda i: (0, 0)),
        ],
        out_specs=[
            pl.BlockSpec((r, h), lambda i: (i, 0)),
            pl.BlockSpec((r, h), lambda i: (i, 0)),
        ],
        out_shape=[
            jax.ShapeDtypeStruct((npad, h), jnp.float32),
            jax.ShapeDtypeStruct((npad, h), jnp.float32),
        ],
    )


def _mm_combine(npad, w_in, w_out, r):
    def body(p_ref, d_ref, z_ref, b_ref, wl_ref, wr_ref, y_ref, z2_ref):
        s = p_ref[0] + p_ref[1]
        deg = d_ref[0, :, 0] + d_ref[1, :, 0]
        dc = jnp.maximum(deg, 1.0)
        hh = s / dc[:, None] + b_ref[...] + z_ref[...]
        hh = jnp.maximum(hh, 0.0)
        y_ref[...] = jnp.dot(hh, wl_ref[...], preferred_element_type=jnp.float32)
        z2_ref[...] = jnp.dot(hh, wr_ref[...], preferred_element_type=jnp.float32)

    return pl.pallas_call(
        body,
        grid=(npad // r,),
        in_specs=[
            pl.BlockSpec((NC, r, w_in), lambda i: (0, i, 0)),
            pl.BlockSpec((NC, r, 128), lambda i: (0, i, 0)),
            pl.BlockSpec((r, w_in), lambda i: (i, 0)),
            pl.BlockSpec((1, w_in), lambda i: (0, 0)),
            pl.BlockSpec((w_in, w_out), lambda i: (0, 0)),
            pl.BlockSpec((w_in, w_out), lambda i: (0, 0)),
        ],
        out_specs=[
            pl.BlockSpec((r, w_out), lambda i: (i, 0)),
            pl.BlockSpec((r, w_out), lambda i: (i, 0)),
        ],
        out_shape=[
            jax.ShapeDtypeStruct((npad, w_out), jnp.float32),
            jax.ShapeDtypeStruct((npad, w_out), jnp.float32),
        ],
    )


def _mm_final(npad, w, r):
    def body(p_ref, d_ref, z_ref, b_ref, o_ref):
        s = p_ref[0] + p_ref[1]
        deg = d_ref[0, :, 0] + d_ref[1, :, 0]
        dc = jnp.maximum(deg, 1.0)
        o_ref[...] = s / dc[:, None] + b_ref[...] + z_ref[...]

    return pl.pallas_call(
        body,
        grid=(npad // r,),
        in_specs=[
            pl.BlockSpec((NC, r, w), lambda i: (0, i, 0)),
            pl.BlockSpec((NC, r, 128), lambda i: (0, i, 0)),
            pl.BlockSpec((r, w), lambda i: (i, 0)),
            pl.BlockSpec((1, w), lambda i: (0, 0)),
        ],
        out_specs=pl.BlockSpec((r, w), lambda i: (i, 0)),
        out_shape=jax.ShapeDtypeStruct((npad, w), jnp.float32),
    )


def kernel(x, edge_index, relations, Wl1, bl1, Wr1, Wl2, bl2, Wr2, Wl3, bl3, Wr3):
    n, d = x.shape
    e = edge_index.shape[1]
    h = Wl1.shape[1]
    c = Wl3.shape[1]
    cpad = 48
    tile = NS * CHUNK
    npad = ((n + tile - 1) // tile) * tile
    if npad == n:
        npad += tile
    r = 1024

    x_pad = jnp.pad(x, ((0, npad - n), (0, 0)))
    nchunks = e // CHUNK
    nw = NC * NS
    ncp = -(-nchunks // (16 * nw)) * (16 * nw)
    ei = edge_index.astype(jnp.int32).reshape(2, nchunks, CHUNK)
    ei = jnp.transpose(ei, (1, 0, 2))
    if ncp > nchunks:
        npc = ncp - nchunks
        ar = jnp.arange(npc * CHUNK, dtype=jnp.int32)
        pad = jnp.stack(
            [(ar % n).reshape(npc, CHUNK),
             (n + ar % (npad - n)).reshape(npc, CHUNK)], axis=1)
        ei = jnp.concatenate([ei, pad], axis=0)
    wl3p = jnp.pad(Wl3, ((0, 0), (0, cpad - c)))
    wr3p = jnp.pad(Wr3, ((0, 0), (0, cpad - c)))
    bl3p = jnp.pad(bl3, (0, cpad - c)).reshape(1, cpad)

    dpart = _make_deg(npad, ncp)(ei)
    y1, z1 = _mm_first(npad, d, h, r)(x_pad, Wl1, Wr1)
    p1 = _make_segsum(npad, ncp, h)(y1, ei)
    y2, z2 = _mm_combine(npad, h, h, r)(p1, dpart, z1, bl1.reshape(1, h), Wl2, Wr2)
    p2 = _make_segsum(npad, ncp, h)(y2, ei)
    y3, z3 = _mm_combine(npad, h, cpad, r)(p2, dpart, z2, bl2.reshape(1, h), wl3p, wr3p)
    p3 = _make_segsum(npad, ncp, cpad)(y3, ei)
    out = _mm_final(npad, cpad, r)(p3, dpart, z3, bl3p)
    return out[:n, :c]


# speedup vs baseline: 3.5785x; 1.0036x over previous
import functools
import jax
import jax.numpy as jnp
from jax import lax
from jax.experimental import pallas as pl
from jax.experimental.pallas import tpu as pltpu
from jax.experimental.pallas import tpu_sc as plsc

NC, NS, LANES = 2, 16, 16
CHUNK = 128


@functools.lru_cache(maxsize=None)
def _make_segsum(npad, n_chunks, w):
    nw = NC * NS
    cpw = n_chunks // nw
    G = 16
    ngroups = cpw // G
    rows_per_tile = npad // NS
    wb_chunks = rows_per_tile // CHUNK
    mesh = plsc.VectorSubcoreMesh(core_axis_name="c", subcore_axis_name="s")

    def body(y_hbm, ei_hbm, p_out, eig0, eig1, rows0, rows1, acc,
             sem0, sem1, gsem, esem0, esem1):
        ci = lax.axis_index("c")
        si = lax.axis_index("s")
        wid = si * NC + ci
        zeros16 = jnp.zeros((LANES,), jnp.float32)
        rows = (rows0, rows1)
        sems = (sem0, sem1)
        eigs = (eig0, eig1)
        esems = (esem0, esem1)

        @pl.loop(0, CHUNK)
        def _zr(r):
            for j in range(w // LANES):
                rows0[r, pl.ds(j * LANES, LANES)] = zeros16

        row0 = si * rows_per_tile

        @pl.loop(0, wb_chunks)
        def _za(k):
            pltpu.sync_copy(rows0, acc.at[pl.ds(row0 + k * CHUNK, CHUNK)])

        def eload(g, b):
            pltpu.async_copy(ei_hbm.at[pl.ds(wid * cpw + g * G, G)],
                             eigs[b], esems[b])

        def ewait(b):
            pltpu.make_async_copy(ei_hbm.at[pl.ds(0, G)], eigs[b],
                                  esems[b]).wait()

        eload(0, 0)
        plsc.subcore_barrier()

        def swait(b):
            pltpu.make_async_copy(rows[b], acc.at[eigs[0].at[0, 1]],
                                  sems[b]).wait()

        for g in range(ngroups):
            eb = g % 2
            ewait(eb)
            if g + 1 < ngroups:
                eload(g + 1, 1 - eb)
            eig = eigs[eb]

            def gath(k, b, eig=eig):
                pltpu.async_copy(y_hbm.at[eig.at[k, 0]], rows[b],
                                 gsem).wait()

            def sstart(k, b, eig=eig):
                pltpu.async_copy(rows[b], acc.at[eig.at[k, 1]], sems[b],
                                 add=True)

            if g == 0:
                gath(0, 0)
                sstart(0, 0)
                gath(1, 1)
                sstart(1, 1)
                k0 = 2
            else:
                k0 = 0

            @pl.loop(0, (G - k0) // 2)
            def _edges(k2):
                for b in (0, 1):
                    k = k0 + k2 * 2 + b
                    swait(b)
                    gath(k, b)
                    sstart(k, b)

        swait(0)
        swait(1)

        plsc.subcore_barrier()

        @pl.loop(0, wb_chunks)
        def _wb(k):
            r0 = row0 + k * CHUNK
            pltpu.sync_copy(acc.at[pl.ds(r0, CHUNK)],
                            p_out.at[ci, pl.ds(r0, CHUNK)])

    params = None
    if w % 128 != 0:
        params = pltpu.CompilerParams(use_tc_tiling_on_sc=False)
    return pl.kernel(
        body,
        out_type=jax.ShapeDtypeStruct((NC, npad, w), jnp.float32),
        mesh=mesh,
        compiler_params=params,
        scratch_types=(
            pltpu.VMEM((G, 2, CHUNK), jnp.int32),
            pltpu.VMEM((G, 2, CHUNK), jnp.int32),
            pltpu.VMEM((CHUNK, w), jnp.float32),
            pltpu.VMEM((CHUNK, w), jnp.float32),
            pltpu.VMEM_SHARED((npad, w), jnp.float32),
            pltpu.SemaphoreType.DMA,
            pltpu.SemaphoreType.DMA,
            pltpu.SemaphoreType.DMA,
            pltpu.SemaphoreType.DMA,
            pltpu.SemaphoreType.DMA,
        ),
    )


@functools.lru_cache(maxsize=None)
def _make_deg(npad, n_chunks):
    w = LANES
    nw = NC * NS
    cpw = n_chunks // nw
    rows_per_tile = npad // NS
    wb_chunks = rows_per_tile // CHUNK
    mesh = plsc.VectorSubcoreMesh(core_axis_name="c", subcore_axis_name="s")

    def body(ei_hbm, d_out, eis_v, rows_v, dacc, sem0, sem1):
        ci = lax.axis_index("c")
        si = lax.axis_index("s")
        wid = si * NC + ci
        zeros16 = jnp.zeros((LANES,), jnp.float32)
        sems = (sem0, sem1)

        @pl.loop(0, CHUNK)
        def _zr(r):
            for j in range(w // LANES):
                rows_v[r, pl.ds(j * LANES, LANES)] = zeros16

        row0 = si * rows_per_tile

        @pl.loop(0, wb_chunks)
        def _za(k):
            pltpu.sync_copy(rows_v, dacc.at[pl.ds(row0 + k * CHUNK, CHUNK)])

        ones16 = jnp.full((LANES,), 1.0, jnp.float32)

        @pl.loop(0, CHUNK)
        def _fo(r):
            rows_v[r, pl.ds(0, LANES)] = ones16

        pltpu.sync_copy(ei_hbm.at[pl.ds(wid * cpw, cpw)], eis_v)

        plsc.subcore_barrier()

        def start(k, b):
            pltpu.async_copy(rows_v, dacc.at[eis_v.at[k, 1]], sems[b],
                             add=True)

        def wait(b):
            pltpu.make_async_copy(rows_v, dacc.at[eis_v.at[0, 1]],
                                  sems[b]).wait()

        start(0, 0)

        @pl.loop(0, cpw // 2 - 1)
        def _edges(k2):
            for b in (0, 1):
                k = k2 * 2 + b
                start(k + 1, 1 - b)
                wait(b)

        start(cpw - 1, 1)
        wait(0)
        wait(1)

        plsc.subcore_barrier()

        @pl.loop(0, wb_chunks)
        def _wb(k):
            r0 = row0 + k * CHUNK
            pltpu.sync_copy(dacc.at[pl.ds(r0, CHUNK)],
                            d_out.at[ci, pl.ds(r0, CHUNK)])

    return pl.kernel(
        body,
        out_type=jax.ShapeDtypeStruct((NC, npad, w), jnp.float32),
        mesh=mesh,
        compiler_params=pltpu.CompilerParams(use_tc_tiling_on_sc=False),
        scratch_types=(
            pltpu.VMEM((cpw, 2, CHUNK), jnp.int32),
            pltpu.VMEM((CHUNK, w), jnp.float32),
            pltpu.VMEM_SHARED((npad, w), jnp.float32),
            pltpu.SemaphoreType.DMA,
            pltpu.SemaphoreType.DMA,
        ),
    )


def _mm_first(npad, d, h, r):
    def body(x_ref, wl_ref, wr_ref, y_ref, z_ref):
        xb = x_ref[...]
        y_ref[...] = jnp.dot(xb, wl_ref[...], preferred_element_type=jnp.float32)
        z_ref[...] = jnp.dot(xb, wr_ref[...], preferred_element_type=jnp.float32)

    return pl.pallas_call(
        body,
        grid=(npad // r,),
        in_specs=[
            pl.BlockSpec((r, d), lambda i: (i, 0)),
            pl.BlockSpec((d, h), lambda i: (0, 0)),
            pl.BlockSpec((d, h), lambda i: (0, 0)),
        ],
        out_specs=[
            pl.BlockSpec((r, h), lambda i: (i, 0)),
            pl.BlockSpec((r, h), lambda i: (i, 0)),
        ],
        out_shape=[
            jax.ShapeDtypeStruct((npad, h), jnp.float32),
            jax.ShapeDtypeStruct((npad, h), jnp.float32),
        ],
    )


def _mm_combine(npad, w_in, w_out, r):
    def body(p_ref, d_ref, z_ref, b_ref, wl_ref, wr_ref, y_ref, z2_ref):
        s = p_ref[0] + p_ref[1]
        deg = d_ref[0, :, 0] + d_ref[1, :, 0]
        dc = jnp.maximum(deg, 1.0)
        hh = s / dc[:, None] + b_ref[...] + z_ref[...]
        hh = jnp.maximum(hh, 0.0)
        y_ref[...] = jnp.dot(hh, wl_ref[...], preferred_element_type=jnp.float32)
        z2_ref[...] = jnp.dot(hh, wr_ref[...], preferred_element_type=jnp.float32)

    return pl.pallas_call(
        body,
        grid=(npad // r,),
        in_specs=[
            pl.BlockSpec((NC, r, w_in), lambda i: (0, i, 0)),
            pl.BlockSpec((NC, r, 128), lambda i: (0, i, 0)),
            pl.BlockSpec((r, w_in), lambda i: (i, 0)),
            pl.BlockSpec((1, w_in), lambda i: (0, 0)),
            pl.BlockSpec((w_in, w_out), lambda i: (0, 0)),
            pl.BlockSpec((w_in, w_out), lambda i: (0, 0)),
        ],
        out_specs=[
            pl.BlockSpec((r, w_out), lambda i: (i, 0)),
            pl.BlockSpec((r, w_out), lambda i: (i, 0)),
        ],
        out_shape=[
            jax.ShapeDtypeStruct((npad, w_out), jnp.float32),
            jax.ShapeDtypeStruct((npad, w_out), jnp.float32),
        ],
    )


def _mm_final(npad, w, r):
    def body(p_ref, d_ref, z_ref, b_ref, o_ref):
        s = p_ref[0] + p_ref[1]
        deg = d_ref[0, :, 0] + d_ref[1, :, 0]
        dc = jnp.maximum(deg, 1.0)
        o_ref[...] = s / dc[:, None] + b_ref[...] + z_ref[...]

    return pl.pallas_call(
        body,
        grid=(npad // r,),
        in_specs=[
            pl.BlockSpec((NC, r, w), lambda i: (0, i, 0)),
            pl.BlockSpec((NC, r, 128), lambda i: (0, i, 0)),
            pl.BlockSpec((r, w), lambda i: (i, 0)),
            pl.BlockSpec((1, w), lambda i: (0, 0)),
        ],
        out_specs=pl.BlockSpec((r, w), lambda i: (i, 0)),
        out_shape=jax.ShapeDtypeStruct((npad, w), jnp.float32),
    )


def kernel(x, edge_index, relations, Wl1, bl1, Wr1, Wl2, bl2, Wr2, Wl3, bl3, Wr3):
    n, d = x.shape
    e = edge_index.shape[1]
    h = Wl1.shape[1]
    c = Wl3.shape[1]
    cpad = 48
    tile = NS * CHUNK
    npad = ((n + tile - 1) // tile) * tile
    if npad == n:
        npad += tile
    r = 1024

    x_pad = jnp.pad(x, ((0, npad - n), (0, 0)))
    nchunks = e // CHUNK
    nw = NC * NS
    ncp = -(-nchunks // (16 * nw)) * (16 * nw)
    ei = edge_index.astype(jnp.int32).reshape(2, nchunks, CHUNK)
    ei = jnp.transpose(ei, (1, 0, 2))
    if ncp > nchunks:
        npc = ncp - nchunks
        ar = jnp.arange(npc * CHUNK, dtype=jnp.int32)
        pad = jnp.stack(
            [(ar % n).reshape(npc, CHUNK),
             (n + ar % (npad - n)).reshape(npc, CHUNK)], axis=1)
        ei = jnp.concatenate([ei, pad], axis=0)
    wl3p = jnp.pad(Wl3, ((0, 0), (0, cpad - c)))
    wr3p = jnp.pad(Wr3, ((0, 0), (0, cpad - c)))
    bl3p = jnp.pad(bl3, (0, cpad - c)).reshape(1, cpad)

    dpart = _make_deg(npad, ncp)(ei)
    y1, z1 = _mm_first(npad, d, h, r)(x_pad, Wl1, Wr1)
    p1 = _make_segsum(npad, ncp, h)(y1, ei)
    y2, z2 = _mm_combine(npad, h, h, r)(p1, dpart, z1, bl1.reshape(1, h), Wl2, Wr2)
    p2 = _make_segsum(npad, ncp, h)(y2, ei)
    y3, z3 = _mm_combine(npad, h, cpad, r)(p2, dpart, z2, bl2.reshape(1, h), wl3p, wr3p)
    p3 = _make_segsum(npad, ncp, cpad)(y3, ei)
    out = _mm_final(npad, cpad, r)(p3, dpart, z3, bl3p)
    return out[:n, :c]
